# baseline (speedup 1.0000x reference)
import functools

import jax
import jax.numpy as jnp
from jax import lax
from jax.experimental import pallas as pl
from jax.experimental.pallas import tpu as pltpu
from jax.experimental.pallas import tpu_sc as plsc

D = 100
DP = 128
T = 7
OUT = 50

_NC = 2
_NS = 16
_NW = _NC * _NS
_CHUNK = 128
_TBLK = 8192


def _tr_body(xT_ref, eye_ref, o_ref):
    o_ref[:, :D] = lax.dot_general(
        xT_ref[...], eye_ref[...], (((0,), (0,)), ((), ())),
        preferred_element_type=jnp.float32).astype(jnp.bfloat16)
    o_ref[:, D:] = jnp.zeros((o_ref.shape[0], DP - D), jnp.bfloat16)


def _densify(word_table_T, V: int):
    eye = jnp.eye(D, dtype=jnp.float32)
    return pl.pallas_call(
        _tr_body,
        grid=((V + _TBLK - 1) // _TBLK,),
        in_specs=[
            pl.BlockSpec((D, _TBLK), lambda i: (0, i)),
            pl.BlockSpec((D, D), lambda i: (0, 0)),
        ],
        out_specs=pl.BlockSpec((_TBLK, DP), lambda i: (i, 0)),
        out_shape=jax.ShapeDtypeStruct((V, DP), jnp.bfloat16),
    )(word_table_T, eye)


def _make_sc_gather(BT: int, V: int):
    per_w = BT // _NW
    nch = per_w // _CHUNK
    assert per_w * _NW == BT and nch * _CHUNK == per_w and nch % 2 == 0

    mesh = plsc.VectorSubcoreMesh(core_axis_name="c", subcore_axis_name="s",
                                  num_cores=_NC, num_subcores=_NS)

    @functools.partial(
        pl.kernel,
        mesh=mesh,
        out_type=jax.ShapeDtypeStruct((BT, DP), jnp.bfloat16),
        scratch_types=[
            pltpu.VMEM((nch, _CHUNK), jnp.int32),
            pltpu.VMEM((2, _CHUNK, DP), jnp.bfloat16),
            pltpu.SemaphoreType.DMA,
            pltpu.SemaphoreType.DMA,
            pltpu.SemaphoreType.DMA,
            pltpu.SemaphoreType.DMA,
        ],
        compiler_params=pltpu.CompilerParams(use_tc_tiling_on_sc=False,
                                             needs_layout_passes=False),
    )
    def sc_gather(idx_hbm, table_hbm, out_hbm, idx_v, rows_v,
                  gsem0, gsem1, osem0, osem1):
        wid = lax.axis_index("s") * _NC + lax.axis_index("c")
        base = wid * per_w
        pltpu.sync_copy(idx_hbm.at[wid], idx_v)

        def gather(j, slot, sem):
            return pltpu.make_async_copy(
                table_hbm.at[idx_v.at[j]], rows_v.at[slot], sem)

        def out_copy(j, slot, sem):
            return pltpu.make_async_copy(
                rows_v.at[slot], out_hbm.at[pl.ds(base + j * _CHUNK, _CHUNK)],
                sem)

        gather(0, 0, gsem0).start()

        def body(i, carry):
            j0 = 2 * i
            j1 = 2 * i + 1
            gather(j0, 0, gsem0).wait()
            out_copy(j0, 0, osem0).start()

            @pl.when(i > 0)
            def _():
                out_copy(j0 - 1, 1, osem1).wait()

            gather(j1, 1, gsem1).start()
            out_copy(j0, 0, osem0).wait()

            @pl.when(j0 + 2 < nch)
            def _():
                gather(j0 + 2, 0, gsem0).start()

            gather(j1, 1, gsem1).wait()
            out_copy(j1, 1, osem1).start()
            return carry

        lax.fori_loop(0, nch // 2, body, 0)
        out_copy(nch - 1, 1, osem1).wait()

    return sc_gather


def _head_body(we_ref, pidx_ref, didx_ref, ptab_ref, dtab_ref, w_ref, b_ref,
               out_ref, pw_ref):
    @pl.when(pl.program_id(0) == 0)
    def _():
        for t in range(T):
            wt = w_ref[t * DP:t * DP + D, :]
            pw_ref[t * OUT:(t + 1) * OUT, :] = jnp.dot(
                ptab_ref[...], wt, preferred_element_type=jnp.float32)
            pw_ref[(T + t) * OUT:(T + t + 1) * OUT, :] = jnp.dot(
                dtab_ref[...], wt, preferred_element_type=jnp.float32)

    bsz = we_ref.shape[0]
    iota = lax.broadcasted_iota(jnp.int32, (bsz, OUT), 1)
    ohs = [(pidx_ref[:, t:t + 1] == iota).astype(jnp.float32)
           for t in range(T)]
    ohs += [(didx_ref[:, t:t + 1] == iota).astype(jnp.float32)
            for t in range(T)]
    oh = jnp.concatenate(ohs, axis=1)

    acc = jnp.dot(we_ref[...].astype(jnp.float32), w_ref[...],
                  preferred_element_type=jnp.float32)
    acc = acc + jnp.dot(oh, pw_ref[...], preferred_element_type=jnp.float32)
    acc = acc + b_ref[0, :]
    m = jnp.max(acc, axis=-1, keepdims=True)
    e = jnp.exp(acc - m)
    out_ref[...] = e / jnp.sum(e, axis=-1, keepdims=True)


def _tc_head(we2d, pos_idx, dep_idx, pos_table, dep_table, W_pad, b_out2d,
             blk: int):
    B = we2d.shape[0]
    grid = (B // blk,)
    return pl.pallas_call(
        _head_body,
        grid=grid,
        in_specs=[
            pl.BlockSpec((blk, T * DP), lambda i: (i, 0)),
            pl.BlockSpec((blk, T), lambda i: (i, 0)),
            pl.BlockSpec((blk, T), lambda i: (i, 0)),
            pl.BlockSpec((OUT, D), lambda i: (0, 0)),
            pl.BlockSpec((OUT, D), lambda i: (0, 0)),
            pl.BlockSpec((T * DP, OUT), lambda i: (0, 0)),
            pl.BlockSpec((1, OUT), lambda i: (0, 0)),
        ],
        out_specs=pl.BlockSpec((blk, OUT), lambda i: (i, 0)),
        out_shape=jax.ShapeDtypeStruct((B, OUT), jnp.float32),
        scratch_shapes=[pltpu.VMEM((2 * T * OUT, OUT), jnp.float32)],
    )(we2d, pos_idx, dep_idx, pos_table, dep_table, W_pad, b_out2d)


def kernel(word_idx, pos_idx, dep_idx, word_table, pos_table, dep_table,
           W_out, b_out):
    B, t = word_idx.shape
    assert t == T
    BT = B * T
    V = word_table.shape[0]

    tbl = _densify(word_table.T, V)

    wi = word_idx.astype(jnp.int32).reshape(_NW, BT // (_NW * _CHUNK), _CHUNK)
    we = _make_sc_gather(BT, V)(wi, tbl)
    we2d = we.reshape(B, T * DP)

    W_pad = jnp.pad(W_out.reshape(T, D, OUT), ((0, 0), (0, DP - D), (0, 0)))
    W_pad = W_pad.reshape(T * DP, OUT)

    return _tc_head(we2d, pos_idx.astype(jnp.int32), dep_idx.astype(jnp.int32),
                    pos_table, dep_table, W_pad, b_out.reshape(1, OUT),
                    blk=1024)


# speedup vs baseline: 2.6183x; 2.6183x over previous
import functools

import jax
import jax.numpy as jnp
from jax import lax
from jax.experimental import pallas as pl
from jax.experimental.pallas import tpu as pltpu
from jax.experimental.pallas import tpu_sc as plsc

D = 100
DP = 128
T = 7
OUT = 50

_NC = 2
_NS = 16
_NW = _NC * _NS
_CHUNK = 128
_TBLK = 8192


def _tr_body(xT_ref, eye_ref, o_ref):
    o_ref[:, :D] = lax.dot_general(
        xT_ref[...], eye_ref[...], (((0,), (0,)), ((), ())),
        preferred_element_type=jnp.float32)
    o_ref[:, D:] = jnp.zeros((o_ref.shape[0], DP - D), jnp.float32)


def _densify(word_table_T, V: int):
    eye = jnp.eye(D, dtype=jnp.float32)
    return pl.pallas_call(
        _tr_body,
        grid=((V + _TBLK - 1) // _TBLK,),
        in_specs=[
            pl.BlockSpec((D, _TBLK), lambda i: (0, i)),
            pl.BlockSpec((D, D), lambda i: (0, 0)),
        ],
        out_specs=pl.BlockSpec((_TBLK, DP), lambda i: (i, 0)),
        out_shape=jax.ShapeDtypeStruct((V, DP), jnp.float32),
    )(word_table_T, eye)


def _make_sc_gather(BT: int, V: int):
    per_w = BT // _NW
    nch = per_w // _CHUNK
    assert per_w * _NW == BT and nch * _CHUNK == per_w and nch % 2 == 0

    mesh = plsc.VectorSubcoreMesh(core_axis_name="c", subcore_axis_name="s",
                                  num_cores=_NC, num_subcores=_NS)

    @functools.partial(
        pl.kernel,
        mesh=mesh,
        out_type=jax.ShapeDtypeStruct((BT, DP), jnp.float32),
        scratch_types=[
            pltpu.VMEM((nch, _CHUNK), jnp.int32),
            pltpu.VMEM((2, _CHUNK, DP), jnp.float32),
            pltpu.SemaphoreType.DMA,
            pltpu.SemaphoreType.DMA,
            pltpu.SemaphoreType.DMA,
            pltpu.SemaphoreType.DMA,
        ],
        compiler_params=pltpu.CompilerParams(use_tc_tiling_on_sc=False,
                                             needs_layout_passes=False),
    )
    def sc_gather(idx_hbm, table_hbm, out_hbm, idx_v, rows_v,
                  gsem0, gsem1, osem0, osem1):
        wid = lax.axis_index("s") * _NC + lax.axis_index("c")
        base = wid * per_w
        pltpu.sync_copy(idx_hbm.at[wid], idx_v)

        def gather(j, slot, sem):
            return pltpu.make_async_copy(
                table_hbm.at[idx_v.at[j]], rows_v.at[slot], sem)

        def out_copy(j, slot, sem):
            return pltpu.make_async_copy(
                rows_v.at[slot], out_hbm.at[pl.ds(base + j * _CHUNK, _CHUNK)],
                sem)

        gather(0, 0, gsem0).start()

        def body(i, carry):
            j0 = 2 * i
            j1 = 2 * i + 1
            gather(j0, 0, gsem0).wait()
            out_copy(j0, 0, osem0).start()

            @pl.when(i > 0)
            def _():
                out_copy(j0 - 1, 1, osem1).wait()

            gather(j1, 1, gsem1).start()
            out_copy(j0, 0, osem0).wait()

            @pl.when(j0 + 2 < nch)
            def _():
                gather(j0 + 2, 0, gsem0).start()

            gather(j1, 1, gsem1).wait()
            out_copy(j1, 1, osem1).start()
            return carry

        lax.fori_loop(0, nch // 2, body, 0)
        out_copy(nch - 1, 1, osem1).wait()

    return sc_gather


def _head_body(we_ref, pidx_ref, didx_ref, ptab_ref, dtab_ref, w_ref, b_ref,
               out_ref, pw_ref):
    @pl.when(pl.program_id(0) == 0)
    def _():
        for t in range(T):
            wt = w_ref[t * DP:t * DP + D, :]
            pw_ref[t * OUT:(t + 1) * OUT, :] = jnp.dot(
                ptab_ref[...], wt, preferred_element_type=jnp.float32)
            pw_ref[(T + t) * OUT:(T + t + 1) * OUT, :] = jnp.dot(
                dtab_ref[...], wt, preferred_element_type=jnp.float32)

    bsz = we_ref.shape[0]
    iota = lax.broadcasted_iota(jnp.int32, (bsz, OUT), 1)
    ohs = [(pidx_ref[:, t:t + 1] == iota).astype(jnp.float32)
           for t in range(T)]
    ohs += [(didx_ref[:, t:t + 1] == iota).astype(jnp.float32)
            for t in range(T)]
    oh = jnp.concatenate(ohs, axis=1)

    acc = jnp.dot(we_ref[...], w_ref[...], preferred_element_type=jnp.float32)
    acc = acc + jnp.dot(oh, pw_ref[...], preferred_element_type=jnp.float32)
    acc = acc + b_ref[0, :]
    m = jnp.max(acc, axis=-1, keepdims=True)
    e = jnp.exp(acc - m)
    out_ref[...] = e / jnp.sum(e, axis=-1, keepdims=True)


def _tc_head(we2d, pos_idx, dep_idx, pos_table, dep_table, W_pad, b_out2d,
             blk: int):
    B = we2d.shape[0]
    grid = (B // blk,)
    return pl.pallas_call(
        _head_body,
        grid=grid,
        in_specs=[
            pl.BlockSpec((blk, T * DP), lambda i: (i, 0)),
            pl.BlockSpec((blk, T), lambda i: (i, 0)),
            pl.BlockSpec((blk, T), lambda i: (i, 0)),
            pl.BlockSpec((OUT, D), lambda i: (0, 0)),
            pl.BlockSpec((OUT, D), lambda i: (0, 0)),
            pl.BlockSpec((T * DP, OUT), lambda i: (0, 0)),
            pl.BlockSpec((1, OUT), lambda i: (0, 0)),
        ],
        out_specs=pl.BlockSpec((blk, OUT), lambda i: (i, 0)),
        out_shape=jax.ShapeDtypeStruct((B, OUT), jnp.float32),
        scratch_shapes=[pltpu.VMEM((2 * T * OUT, OUT), jnp.float32)],
    )(we2d, pos_idx, dep_idx, pos_table, dep_table, W_pad, b_out2d)


def kernel(word_idx, pos_idx, dep_idx, word_table, pos_table, dep_table,
           W_out, b_out):
    B, t = word_idx.shape
    assert t == T
    BT = B * T
    V = word_table.shape[0]

    tbl = _densify(word_table.T, V)

    wi = word_idx.astype(jnp.int32).reshape(_NW, BT // (_NW * _CHUNK), _CHUNK)
    we = _make_sc_gather(BT, V)(wi, tbl)
    we2d = we.reshape(B, T * DP)

    W_pad = jnp.pad(W_out.reshape(T, D, OUT), ((0, 0), (0, DP - D), (0, 0)))
    W_pad = W_pad.reshape(T * DP, OUT)

    return _tc_head(we2d, pos_idx.astype(jnp.int32), dep_idx.astype(jnp.int32),
                    pos_table, dep_table, W_pad, b_out.reshape(1, OUT),
                    blk=1024)


# speedup vs baseline: 2.6836x; 1.0250x over previous
import functools

import jax
import jax.numpy as jnp
from jax import lax
from jax.experimental import pallas as pl
from jax.experimental.pallas import tpu as pltpu
from jax.experimental.pallas import tpu_sc as plsc

D = 100
DP = 128
T = 7
OUT = 50

_NC = 2
_NS = 16
_NW = _NC * _NS
_CHUNK = 128
_TBLK = 16384


def _tr_body(xT_ref, eye_ref, o_ref):
    o_ref[:, :D] = lax.dot_general(
        xT_ref[...], eye_ref[...], (((0,), (0,)), ((), ())),
        preferred_element_type=jnp.float32)
    o_ref[:, D:] = jnp.zeros((o_ref.shape[0], DP - D), jnp.float32)


def _densify(word_table_T, V: int):
    eye = jnp.eye(D, dtype=jnp.float32)
    return pl.pallas_call(
        _tr_body,
        grid=((V + _TBLK - 1) // _TBLK,),
        in_specs=[
            pl.BlockSpec((D, _TBLK), lambda i: (0, i)),
            pl.BlockSpec((D, D), lambda i: (0, 0)),
        ],
        out_specs=pl.BlockSpec((_TBLK, DP), lambda i: (i, 0)),
        out_shape=jax.ShapeDtypeStruct((V, DP), jnp.float32),
    )(word_table_T, eye)


def _make_sc_gather(BT: int, V: int):
    per_w = BT // _NW
    nch = per_w // _CHUNK
    assert per_w * _NW == BT and nch * _CHUNK == per_w and nch % 2 == 0

    mesh = plsc.VectorSubcoreMesh(core_axis_name="c", subcore_axis_name="s",
                                  num_cores=_NC, num_subcores=_NS)

    @functools.partial(
        pl.kernel,
        mesh=mesh,
        out_type=jax.ShapeDtypeStruct((BT, DP), jnp.float32),
        scratch_types=[
            pltpu.VMEM((nch, _CHUNK), jnp.int32),
            pltpu.VMEM((2, _CHUNK, DP), jnp.float32),
            pltpu.SemaphoreType.DMA,
            pltpu.SemaphoreType.DMA,
            pltpu.SemaphoreType.DMA,
            pltpu.SemaphoreType.DMA,
        ],
        compiler_params=pltpu.CompilerParams(use_tc_tiling_on_sc=False,
                                             needs_layout_passes=False),
    )
    def sc_gather(idx_hbm, table_hbm, out_hbm, idx_v, rows_v,
                  gsem0, gsem1, osem0, osem1):
        wid = lax.axis_index("s") * _NC + lax.axis_index("c")
        base = wid * per_w
        pltpu.sync_copy(idx_hbm.at[wid], idx_v)

        def gather(j, slot, sem):
            return pltpu.make_async_copy(
                table_hbm.at[idx_v.at[j]], rows_v.at[slot], sem)

        def out_copy(j, slot, sem):
            return pltpu.make_async_copy(
                rows_v.at[slot], out_hbm.at[pl.ds(base + j * _CHUNK, _CHUNK)],
                sem)

        gather(0, 0, gsem0).start()

        def body(i, carry):
            j0 = 2 * i
            j1 = 2 * i + 1
            gather(j0, 0, gsem0).wait()
            out_copy(j0, 0, osem0).start()

            @pl.when(i > 0)
            def _():
                out_copy(j0 - 1, 1, osem1).wait()

            gather(j1, 1, gsem1).start()
            out_copy(j0, 0, osem0).wait()

            @pl.when(j0 + 2 < nch)
            def _():
                gather(j0 + 2, 0, gsem0).start()

            gather(j1, 1, gsem1).wait()
            out_copy(j1, 1, osem1).start()
            return carry

        lax.fori_loop(0, nch // 2, body, 0)
        out_copy(nch - 1, 1, osem1).wait()

    return sc_gather


def _head_body(we_ref, pidx_ref, didx_ref, ptab_ref, dtab_ref, w_ref, b_ref,
               out_ref, pw_ref):
    @pl.when(pl.program_id(0) == 0)
    def _():
        for t in range(T):
            wt = w_ref[t * DP:t * DP + D, :]
            pw_ref[t * OUT:(t + 1) * OUT, :] = jnp.dot(
                ptab_ref[...], wt, preferred_element_type=jnp.float32)
            pw_ref[(T + t) * OUT:(T + t + 1) * OUT, :] = jnp.dot(
                dtab_ref[...], wt, preferred_element_type=jnp.float32)

    bsz = we_ref.shape[0]
    iota = lax.broadcasted_iota(jnp.int32, (bsz, OUT), 1)
    ohs = [(pidx_ref[:, t:t + 1] == iota).astype(jnp.float32)
           for t in range(T)]
    ohs += [(didx_ref[:, t:t + 1] == iota).astype(jnp.float32)
            for t in range(T)]
    oh = jnp.concatenate(ohs, axis=1)

    acc = jnp.dot(we_ref[...], w_ref[...], preferred_element_type=jnp.float32)
    acc = acc + jnp.dot(oh, pw_ref[...], preferred_element_type=jnp.float32)
    acc = acc + b_ref[0, :]
    m = jnp.max(acc, axis=-1, keepdims=True)
    e = jnp.exp(acc - m)
    out_ref[...] = e / jnp.sum(e, axis=-1, keepdims=True)


def _tc_head(we2d, pos_idx, dep_idx, pos_table, dep_table, W_pad, b_out2d,
             blk: int):
    B = we2d.shape[0]
    grid = (B // blk,)
    return pl.pallas_call(
        _head_body,
        grid=grid,
        in_specs=[
            pl.BlockSpec((blk, T * DP), lambda i: (i, 0)),
            pl.BlockSpec((blk, T), lambda i: (i, 0)),
            pl.BlockSpec((blk, T), lambda i: (i, 0)),
            pl.BlockSpec((OUT, D), lambda i: (0, 0)),
            pl.BlockSpec((OUT, D), lambda i: (0, 0)),
            pl.BlockSpec((T * DP, OUT), lambda i: (0, 0)),
            pl.BlockSpec((1, OUT), lambda i: (0, 0)),
        ],
        out_specs=pl.BlockSpec((blk, OUT), lambda i: (i, 0)),
        out_shape=jax.ShapeDtypeStruct((B, OUT), jnp.float32),
        scratch_shapes=[pltpu.VMEM((2 * T * OUT, OUT), jnp.float32)],
    )(we2d, pos_idx, dep_idx, pos_table, dep_table, W_pad, b_out2d)


def kernel(word_idx, pos_idx, dep_idx, word_table, pos_table, dep_table,
           W_out, b_out):
    B, t = word_idx.shape
    assert t == T
    BT = B * T
    V = word_table.shape[0]

    tbl = _densify(word_table.T, V)

    wi = word_idx.astype(jnp.int32).reshape(_NW, BT // (_NW * _CHUNK), _CHUNK)
    we = _make_sc_gather(BT, V)(wi, tbl)
    we2d = we.reshape(B, T * DP)

    W_pad = jnp.pad(W_out.reshape(T, D, OUT), ((0, 0), (0, DP - D), (0, 0)))
    W_pad = W_pad.reshape(T * DP, OUT)

    return _tc_head(we2d, pos_idx.astype(jnp.int32), dep_idx.astype(jnp.int32),
                    pos_table, dep_table, W_pad, b_out.reshape(1, OUT),
                    blk=1024)
